# Initial kernel scaffold; baseline (speedup 1.0000x reference)
import functools

import jax
import jax.numpy as jnp
from jax import lax
from jax.experimental import pallas as pl
from jax.experimental.pallas import tpu as pltpu
from jax.experimental.pallas import tpu_sc as plsc

NC = 2
NS = 16
NW = NC * NS
L = 16
NBITS = 11
RADIX = 1 << NBITS
SHIFTS = (0, NBITS, 2 * NBITS)
INT_MAX = jnp.int32(2**31 - 1)
INT_MIN = jnp.int32(-(2**31))


def _mesh():
  return plsc.VectorSubcoreMesh(core_axis_name="c", subcore_axis_name="s")


def _wid():
  return lax.axis_index("c") * NS + lax.axis_index("s")


def _lane():
  return lax.iota(jnp.int32, L)


def _bfull(x, dtype=jnp.int32):
  return jnp.full((L,), x, dtype)


def _bcast(vref, idx):
  return plsc.load_gather(vref, [_bfull(idx)])


def _floor_div(p, leafpat):
  q = p / leafpat
  t = q.astype(jnp.int32)
  tf = t.astype(jnp.float32)
  return jnp.where(tf > q, t - 1, t)


def _k1_minmax(n):
  c = n // NW
  wf = 12288
  nwin = (c * 3) // wf

  def body(pts, leaf, mm, pv, lv, rowv):
    wid = _wid()
    lane = _lane()
    pltpu.sync_copy(leaf, lv)
    leafpat = [plsc.load_gather(lv, [(lane + j) % 3]) for j in range(3)]
    acc0 = []
    for _ in range(3):
      acc0 += [_bfull(INT_MAX), _bfull(INT_MIN)]

    def win_loop(w, acc):
      base = wid * (c * 3) + w * wf
      pltpu.sync_copy(pts.at[pl.ds(base, wf)], pv)

      def step(i, acc):
        acc = list(acc)
        for j in range(3):
          p = pv[pl.ds(48 * i + 16 * j, L)]
          cc = _floor_div(p, leafpat[j])
          acc[2 * j] = jnp.minimum(acc[2 * j], cc)
          acc[2 * j + 1] = jnp.maximum(acc[2 * j + 1], cc)
        return tuple(acc)

      return lax.fori_loop(0, wf // 48, step, acc)

    acc = lax.fori_loop(0, nwin, win_loop, tuple(acc0))
    row = _bfull(0)
    for m in range(3):
      vmin = _bfull(INT_MAX)
      vmax = _bfull(INT_MIN)
      for j in range(3):
        cm = (lane + j) % 3 == m
        vmin = jnp.minimum(vmin, jnp.where(cm, acc[2 * j], _bfull(INT_MAX)))
        vmax = jnp.maximum(vmax, jnp.where(cm, acc[2 * j + 1],
                                           _bfull(INT_MIN)))
      smin = jnp.min(vmin)
      smax = jnp.max(vmax)
      row = jnp.where(lane == m, _bfull(smin), row)
      row = jnp.where(lane == m + 3, _bfull(smax), row)
    rowv[...] = row
    pltpu.sync_copy(rowv, mm.at[pl.ds(wid * L, L)])

  return pl.kernel(
      body,
      out_type=jax.ShapeDtypeStruct((NW * L,), jnp.int32),
      mesh=_mesh(),
      scratch_types=[
          pltpu.VMEM((wf,), jnp.float32),
          pltpu.VMEM((L,), jnp.float32),
          pltpu.VMEM((L,), jnp.int32),
      ],
  )


def _global_minmax(mmv, lane):
  acc = jnp.where(lane < 3, _bfull(INT_MAX), _bfull(INT_MIN))

  def step(w, acc):
    row = mmv[pl.ds(L * w, L)]
    return jnp.where(lane < 3, jnp.minimum(acc, row), jnp.maximum(acc, row))

  return lax.fori_loop(0, NW, step, acc)


def _k2_keys_hist(n):
  c = n // NW
  wpt = 4096
  wf = wpt * 3
  nwin = c // wpt

  def body(pts, leaf, mm, keys, hist, pv, lv, mmv, redv, cvec, kv, histv,
           stage):
    wid = _wid()
    lane = _lane()
    pltpu.sync_copy(leaf, lv)
    pltpu.sync_copy(mm, mmv)
    leafpat = [plsc.load_gather(lv, [(lane + j) % 3]) for j in range(3)]
    redv[...] = _global_minmax(mmv, lane)
    mn = [_bcast(redv, m) for m in range(3)]
    mx = [_bcast(redv, m + 3) for m in range(3)]
    d2 = mx[2] - mn[2] + 1
    d1 = mx[1] - mn[1] + 1
    d12 = d1 * d2
    wvec = jnp.where(lane == 0, d12, jnp.where(lane == 1, d2, _bfull(1)))
    mvec = jnp.where(lane == 0, mn[0], jnp.where(lane == 1, mn[1], mn[2]))
    cvec[pl.ds(0, L)] = wvec
    cvec[pl.ds(L, L)] = mvec
    wpat = [plsc.load_gather(cvec, [(lane + j) % 3]) for j in range(3)]
    mpat = [plsc.load_gather(cvec, [L + (lane + j) % 3]) for j in range(3)]

    def zstep(i, _):
      histv[pl.ds(L * i, L)] = _bfull(0)
      return 0

    lax.fori_loop(0, RADIX // L, zstep, 0)

    for win in range(nwin):
      base = wid * c + win * wpt
      pltpu.sync_copy(pts.at[pl.ds(3 * base, wf)], pv)

      def step(i, _):
        for j in range(3):
          p = pv[pl.ds(48 * i + 16 * j, L)]
          cc = _floor_div(p, leafpat[j])
          stage[pl.ds(16 * j, L)] = (cc - mpat[j]) * wpat[j]
        k16 = (plsc.load_gather(stage, [3 * lane]) +
               plsc.load_gather(stage, [3 * lane + 1]) +
               plsc.load_gather(stage, [3 * lane + 2]))
        kv[pl.ds(16 * i, L)] = k16
        d16 = k16 & (RADIX - 1)
        cnt, lastm = plsc.scan_count(d16)
        plsc.addupdate_scatter(histv, [d16], cnt, mask=lastm)
        return 0

      lax.fori_loop(0, wpt // 16, step, 0)
      pltpu.sync_copy(kv, keys.at[pl.ds(base, wpt)])
    pltpu.sync_copy(histv, hist.at[pl.ds(wid * RADIX, RADIX)])

  return pl.kernel(
      body,
      out_type=(jax.ShapeDtypeStruct((n,), jnp.int32),
                jax.ShapeDtypeStruct((NW * RADIX,), jnp.int32)),
      mesh=_mesh(),
      scratch_types=[
          pltpu.VMEM((wf,), jnp.float32),
          pltpu.VMEM((L,), jnp.float32),
          pltpu.VMEM((NW * L,), jnp.int32),
          pltpu.VMEM((L,), jnp.int32),
          pltpu.VMEM((2 * L,), jnp.int32),
          pltpu.VMEM((wpt,), jnp.int32),
          pltpu.VMEM((RADIX,), jnp.int32),
          pltpu.VMEM((48,), jnp.int32),
      ],
  )


def _k_hist(n, shift):
  c = n // NW
  wpt = 8192
  nwin = c // wpt

  def body(keys, hist, kv, histv):
    wid = _wid()
    sh = _bfull(shift)

    def zstep(i, _):
      histv[pl.ds(L * i, L)] = _bfull(0)
      return 0

    lax.fori_loop(0, RADIX // L, zstep, 0)
    for win in range(nwin):
      base = wid * c + win * wpt
      pltpu.sync_copy(keys.at[pl.ds(base, wpt)], kv)

      def step(i, _):
        k16 = kv[pl.ds(16 * i, L)]
        d16 = lax.shift_right_logical(k16, sh) & (RADIX - 1)
        cnt, lastm = plsc.scan_count(d16)
        plsc.addupdate_scatter(histv, [d16], cnt, mask=lastm)
        return 0

      lax.fori_loop(0, wpt // 16, step, 0)
    pltpu.sync_copy(histv, hist.at[pl.ds(wid * RADIX, RADIX)])

  return pl.kernel(
      body,
      out_type=jax.ShapeDtypeStruct((NW * RADIX,), jnp.int32),
      mesh=_mesh(),
      scratch_types=[
          pltpu.VMEM((wpt,), jnp.int32),
          pltpu.VMEM((RADIX,), jnp.int32),
      ],
  )


def _base_offsets(histm, basev, wid, lane):
  wid_b = _bfull(wid)

  def dstep(dc, carry):
    tot = _bfull(0)
    below = _bfull(0)
    for w in range(NW):
      row = histm[pl.ds(w * RADIX + L * dc, L)]
      tot = tot + row
      below = below + jnp.where(_bfull(w) < wid_b, row, _bfull(0))
    incl = plsc.cumsum(tot)
    basev[pl.ds(L * dc, L)] = incl - tot + carry + below
    return carry + _bfull(jnp.sum(tot))

  lax.fori_loop(0, RADIX // L, dstep, _bfull(0))


def _k_permute(n, shift, first):
  c = n // NW
  wpt = 4096
  wf = wpt * 3
  nwin = c // wpt
  f32 = jnp.float32

  def body(keys, hist, *rest):
    if first:
      (pts, keys_o, x_o, y_o, z_o,
       histm, basev, kv, xv, yv, zv, posv, pv, stage, sem) = rest
    else:
      (x_i, y_i, z_i, keys_o, x_o, y_o, z_o,
       histm, basev, kv, xv, yv, zv, posv, sem) = rest
    wid = _wid()
    lane = _lane()
    sh = _bfull(shift)
    pltpu.sync_copy(hist, histm)
    _base_offsets(histm, basev, wid, lane)

    for win in range(nwin):
      base = wid * c + win * wpt
      pltpu.sync_copy(keys.at[pl.ds(base, wpt)], kv)
      if first:
        pltpu.sync_copy(pts.at[pl.ds(3 * base, wf)], pv)

        def tstep(i, _):
          stage[pl.ds(0, L)] = pv[pl.ds(48 * i, L)]
          stage[pl.ds(L, L)] = pv[pl.ds(48 * i + 16, L)]
          stage[pl.ds(2 * L, L)] = pv[pl.ds(48 * i + 32, L)]
          xv[pl.ds(16 * i, L)] = plsc.load_gather(stage, [3 * lane])
          yv[pl.ds(16 * i, L)] = plsc.load_gather(stage, [3 * lane + 1])
          zv[pl.ds(16 * i, L)] = plsc.load_gather(stage, [3 * lane + 2])
          return 0

        lax.fori_loop(0, wpt // 16, tstep, 0)
      else:
        pltpu.sync_copy(x_i.at[pl.ds(base, wpt)], xv)
        pltpu.sync_copy(y_i.at[pl.ds(base, wpt)], yv)
        pltpu.sync_copy(z_i.at[pl.ds(base, wpt)], zv)

      def step(i, _):
        k16 = kv[pl.ds(16 * i, L)]
        d16 = lax.shift_right_logical(k16, sh) & (RADIX - 1)
        cnt, lastm = plsc.scan_count(d16)
        cur = plsc.load_gather(basev, [d16])
        posv[pl.ds(16 * i, L)] = cur + cnt - 1
        plsc.addupdate_scatter(basev, [d16], cnt, mask=lastm)
        return 0

      lax.fori_loop(0, wpt // 16, step, 0)
      cps = [pltpu.async_copy(kv, keys_o.at[posv], sem),
             pltpu.async_copy(xv, x_o.at[posv], sem),
             pltpu.async_copy(yv, y_o.at[posv], sem),
             pltpu.async_copy(zv, z_o.at[posv], sem)]
      for cp in cps:
        cp.wait()

  scratch = [
      pltpu.VMEM((NW * RADIX,), jnp.int32),
      pltpu.VMEM((RADIX,), jnp.int32),
      pltpu.VMEM((wpt,), jnp.int32),
      pltpu.VMEM((wpt,), f32),
      pltpu.VMEM((wpt,), f32),
      pltpu.VMEM((wpt,), f32),
      pltpu.VMEM((wpt,), jnp.int32),
  ]
  if first:
    scratch += [pltpu.VMEM((wf,), f32), pltpu.VMEM((48,), f32)]
  scratch += [pltpu.SemaphoreType.DMA]
  return pl.kernel(
      body,
      out_type=(jax.ShapeDtypeStruct((n,), jnp.int32),
                jax.ShapeDtypeStruct((n,), f32),
                jax.ShapeDtypeStruct((n,), f32),
                jax.ShapeDtypeStruct((n,), f32)),
      mesh=_mesh(),
      scratch_types=scratch,
  )


def _k8a_bounds(n):
  c = n // NW
  wpt = 8192
  nwin = c // wpt

  def body(keys, bc, kpad, prevv, rowv):
    wid = _wid()
    lane = _lane()
    prevv[...] = _bfull(-1)

    @pl.when(wid > 0)
    def _():
      pltpu.sync_copy(keys.at[pl.ds(wid * c - L, L)], prevv)

    prev_b = _bcast(prevv, L - 1)
    plsc.store_scatter(kpad, [_bfull(15)], prev_b, mask=lane == 0)
    acc = _bfull(0)
    for win in range(nwin):
      base = wid * c + win * wpt
      pltpu.sync_copy(keys.at[pl.ds(base, wpt)], kpad.at[pl.ds(L, wpt)])

      def step(i, acc):
        k16 = kpad[pl.ds(L + 16 * i, L)]
        p16 = plsc.load_gather(kpad, [15 + 16 * i + lane])
        return acc + jnp.where(k16 != p16, 1, 0)

      acc = lax.fori_loop(0, wpt // 16, step, acc)
      lastk = _bcast(kpad, L + wpt - 1)
      plsc.store_scatter(kpad, [_bfull(15)], lastk, mask=lane == 0)
    rowv[...] = jnp.where(lane == 0, _bfull(jnp.sum(acc)), _bfull(0))
    pltpu.sync_copy(rowv, bc.at[pl.ds(wid * L, L)])

  return pl.kernel(
      body,
      out_type=jax.ShapeDtypeStruct((NW * L,), jnp.int32),
      mesh=_mesh(),
      scratch_types=[
          pltpu.VMEM((wpt + L,), jnp.int32),
          pltpu.VMEM((L,), jnp.int32),
          pltpu.VMEM((L,), jnp.int32),
      ],
  )


def _k8b_segsum(n):
  c = n // NW
  wpt = 4096
  nwin = c // wpt
  nrow = wpt + L
  nsum = n + nrow + L
  f32 = jnp.float32

  def body(keys, xs, ys, zs, bc, sums, meta, kpad, xv, yv, zv, resv, idxv,
           bcv, sxv, metav, tmpv, sem):
    wid = _wid()
    lane = _lane()
    pltpu.sync_copy(bc, bcv)
    wid_b = _bfull(wid)
    gbase = _bfull(0)
    for w in range(NW):
      cb = _bcast(bcv, L * w)
      gbase = gbase + jnp.where(_bfull(w) < wid_b, cb, _bfull(0))

    tmpv[...] = _bfull(-1)

    @pl.when(wid > 0)
    def _():
      pltpu.sync_copy(keys.at[pl.ds(wid * c - L, L)], tmpv)

    prev_b = _bcast(tmpv, L - 1)
    plsc.store_scatter(kpad, [_bfull(15)], prev_b, mask=lane == 0)
    sxv[pl.ds(0, L)] = jnp.zeros((L,), f32)
    sxv[pl.ds(L, L)] = jnp.zeros((L,), f32)

    zero_f = jnp.zeros((L,), f32)
    gcur = gbase
    opengid = gbase - 1
    ocx = zero_f
    ocy = zero_f
    ocz = zero_f
    occ = zero_f
    anyend = _bfull(0)

    for win in range(nwin):
      base = wid * c + win * wpt
      pltpu.sync_copy(keys.at[pl.ds(base, wpt)], kpad.at[pl.ds(L, wpt)])
      pltpu.sync_copy(xs.at[pl.ds(base, wpt)], xv)
      pltpu.sync_copy(ys.at[pl.ds(base, wpt)], yv)
      pltpu.sync_copy(zs.at[pl.ds(base, wpt)], zv)
      tmpv[...] = _bfull(-2)
      if win < nwin - 1:
        pltpu.sync_copy(keys.at[pl.ds(base + wpt, L)], tmpv)
      else:

        @pl.when(wid < NW - 1)
        def _():
          pltpu.sync_copy(keys.at[pl.ds(base + wpt, L)], tmpv)

      nxt_b = _bcast(tmpv, 0)
      plsc.store_scatter(kpad, [_bfull(L + wpt)], nxt_b, mask=lane == 0)
      winfirst = opengid

      carry0 = (gcur, opengid, ocx, ocy, ocz, occ,
                _bfull(nrow + L), _bfull(-1), anyend)

      def step(i, carry):
        gcur, opengid, ocx, ocy, ocz, occ, rmin, rmax, anyend = carry
        k16 = kpad[pl.ds(L + 16 * i, L)]
        p16 = plsc.load_gather(kpad, [15 + 16 * i + lane])
        n16 = plsc.load_gather(kpad, [17 + 16 * i + lane])
        nb = k16 != p16
        ge = k16 != n16
        nbi = jnp.where(nb, 1, 0)
        incl = plsc.cumsum(nbi)
        gid16 = gcur + incl - 1
        cnt, _lm = plsc.scan_count(gid16)
        firstg = gid16 == opengid
        wloc = gid16 - winfirst
        lge = _bfull(jnp.max(jnp.where(ge, lane, _bfull(-1))))
        nge = jnp.sum(jnp.where(ge, 1, 0))
        hasge = _bfull(nge) > 0
        outs = []
        for comp, (vals, oc) in enumerate(
            ((xv, ocx), (yv, ocy), (zv, ocz), (None, occ))):
          if vals is None:
            tot16 = cnt.astype(f32) + jnp.where(firstg, oc, zero_f)
            sfx = (_bfull(15) - lge).astype(f32)
          else:
            v16 = vals[pl.ds(16 * i, L)]
            s16 = plsc.cumsum(v16)
            sxv[pl.ds(8, L)] = s16
            sstart = plsc.load_gather(sxv, [8 + lane - cnt])
            tot16 = s16 - sstart + jnp.where(firstg, oc, zero_f)
            s15 = _bfull(jnp.sum(v16), f32)
            sfx = s15 - plsc.load_gather(sxv, [8 + lge])
          plsc.store_scatter(resv, [wloc, _bfull(comp)], tot16, mask=ge)
          outs.append(jnp.where(hasge, sfx, oc + sfx))
        ocx, ocy, ocz, occ = outs
        rmin = jnp.minimum(rmin,
                           _bfull(jnp.min(jnp.where(ge, wloc,
                                                    _bfull(nrow + L)))))
        rmax = jnp.maximum(rmax,
                           _bfull(jnp.max(jnp.where(ge, wloc, _bfull(-1)))))
        anyend = anyend | jnp.where(hasge, 1, 0)
        gcur = gcur + _bfull(jnp.sum(nbi))
        opengid = gcur - 1
        return (gcur, opengid, ocx, ocy, ocz, occ, rmin, rmax, anyend)

      (gcur, opengid, ocx, ocy, ocz, occ, rmin, rmax, anyend) = (
          lax.fori_loop(0, wpt // 16, step, carry0))

      def istep(i, _):
        r16 = 16 * i + lane
        valid = (r16 >= rmin) & (r16 <= rmax)
        idxv[pl.ds(16 * i, L)] = jnp.where(valid, winfirst + r16,
                                           _bfull(n) + r16)
        return 0

      lax.fori_loop(0, nrow // 16, istep, 0)
      pltpu.async_copy(resv, sums.at[idxv], sem).wait()
      lastk = _bcast(kpad, L + wpt - 1)
      plsc.store_scatter(kpad, [_bfull(15)], lastk, mask=lane == 0)

    metarow = jnp.where(lane == 0, ocx,
               jnp.where(lane == 1, ocy,
                jnp.where(lane == 2, ocz,
                 jnp.where(lane == 3, occ,
                  jnp.where(lane == 4, anyend.astype(f32), zero_f)))))
    metav[...] = metarow
    pltpu.sync_copy(metav, meta.at[pl.ds(wid * L, L)])

  return pl.kernel(
      body,
      out_type=(jax.ShapeDtypeStruct((nsum, 8), f32),
                jax.ShapeDtypeStruct((NW * L,), f32)),
      mesh=_mesh(),
      scratch_types=[
          pltpu.VMEM((wpt + 2 * L,), jnp.int32),
          pltpu.VMEM((wpt,), f32),
          pltpu.VMEM((wpt,), f32),
          pltpu.VMEM((wpt,), f32),
          pltpu.VMEM((nrow, 8), f32),
          pltpu.VMEM((nrow,), jnp.int32),
          pltpu.VMEM((NW * L,), jnp.int32),
          pltpu.VMEM((2 * L,), f32),
          pltpu.VMEM((L,), f32),
          pltpu.VMEM((L,), jnp.int32),
          pltpu.SemaphoreType.DMA,
      ],
  )


def _k9_means(n, nsum):
  c = n // NW
  wrow = 2048
  nwin = c // wrow
  f32 = jnp.float32

  def body(sums, bc, meta, out, sumv, outv, bcv, tmv, cgv, cvv):
    wid = _wid()
    lane = _lane()
    pltpu.sync_copy(bc, bcv)
    pltpu.sync_copy(meta, tmv)
    zero_f = jnp.zeros((L,), f32)
    ng = _bfull(0)
    for w in range(NW):
      ng = ng + _bcast(bcv, L * w)
    carry = zero_f
    gb = _bfull(0)
    for w in range(NW):
      m16 = tmv[pl.ds(L * w, L)]
      trail = jnp.where(lane < 4, m16, zero_f)
      he = plsc.load_gather(tmv, [_bfull(L * w + 4)]) > 0.5
      tg = gb - 1
      cgv[pl.ds(L * w, L)] = jnp.where(he & (tg >= 0), tg, _bfull(nsum - L))
      cvv[pl.ds(L * w, L)] = jnp.where(he, carry, zero_f)
      carry = jnp.where(he, trail, carry + trail)
      gb = gb + _bcast(bcv, L * w)

    col = jnp.minimum(lane, 7)
    for win in range(nwin):
      rowbase = wid * c + win * wrow
      pltpu.sync_copy(sums.at[pl.ds(rowbase, wrow)], sumv)
      rb = _bfull(rowbase)
      for w in range(NW):
        tg = _bcast(cgv, L * w)
        rel = tg - rb
        inwin = (rel >= 0) & (rel < wrow) & (lane < 8)
        vals = jnp.where(lane < 4, cvv[pl.ds(L * w, L)], zero_f)
        plsc.addupdate_scatter(sumv, [jnp.maximum(rel, 0), col], vals,
                               mask=inwin)

      def ostep(i, _):
        for j in range(3):
          a = 16 * j + lane
          relrow = _bfull(16 * i) + a // 3
          comp = a % 3
          val = plsc.load_gather(sumv, [relrow, comp])
          cntv = plsc.load_gather(sumv, [relrow, _bfull(3)])
          valid = (rb + relrow) < ng
          outv[pl.ds(48 * i + 16 * j, L)] = jnp.where(valid, val / cntv,
                                                      zero_f)
        return 0

      lax.fori_loop(0, wrow // 16, ostep, 0)
      pltpu.sync_copy(outv, out.at[pl.ds(3 * rowbase, 3 * wrow)])

  return pl.kernel(
      body,
      out_type=jax.ShapeDtypeStruct((3 * n,), f32),
      mesh=_mesh(),
      scratch_types=[
          pltpu.VMEM((wrow, 8), f32),
          pltpu.VMEM((3 * wrow,), f32),
          pltpu.VMEM((NW * L,), jnp.int32),
          pltpu.VMEM((NW * L,), f32),
          pltpu.VMEM((NW * L,), jnp.int32),
          pltpu.VMEM((NW * L,), f32),
      ],
  )


def kernel(points, leaf_size):
  n = points.shape[0]
  pts_flat = points.reshape(-1)
  leaf16 = jnp.concatenate(
      [leaf_size.astype(jnp.float32),
       jnp.ones((13,), jnp.float32)])

  mm = _k1_minmax(n)(pts_flat, leaf16)
  keys, hist0 = _k2_keys_hist(n)(pts_flat, leaf16, mm)
  k1s, x1, y1, z1 = _k_permute(n, SHIFTS[0], True)(keys, hist0, pts_flat)
  h1 = _k_hist(n, SHIFTS[1])(k1s)
  k2s, x2, y2, z2 = _k_permute(n, SHIFTS[1], False)(k1s, h1, x1, y1, z1)
  h2 = _k_hist(n, SHIFTS[2])(k2s)
  k3s, x3, y3, z3 = _k_permute(n, SHIFTS[2], False)(k2s, h2, x2, y2, z2)
  bc = _k8a_bounds(n)(k3s)
  sums, meta = _k8b_segsum(n)(k3s, x3, y3, z3, bc)
  nsum = sums.shape[0]
  outflat = _k9_means(n, nsum)(sums, bc, meta)
  ngroups = jnp.sum(bc.reshape(NW, L)[:, 0])
  out = outflat.reshape(n, 3)
  mask = jnp.arange(n) < ngroups
  return (out, mask)


# speedup vs baseline: 13.1393x; 13.1393x over previous
import functools

import jax
import jax.numpy as jnp
from jax import lax
from jax.experimental import pallas as pl
from jax.experimental.pallas import tpu as pltpu
from jax.experimental.pallas import tpu_sc as plsc

NC = 2
NS = 16
NW = NC * NS
L = 16
NBITS = 11
RADIX = 1 << NBITS
SHIFTS = (0, NBITS, 2 * NBITS)
INT_MAX = 2**31 - 1
INT_MIN = -(2**31)


def _c(x):
  return jnp.int32(x)


def _loop(n, body, carry):
  def wrap(_, c):
    i, inner = c
    return (i + _c(1), body(i, inner))

  return lax.fori_loop(0, n, wrap, (_c(0), carry))[1]


def _mesh():
  return plsc.VectorSubcoreMesh(core_axis_name="c", subcore_axis_name="s")


def _wid():
  return lax.axis_index("c") * NS + lax.axis_index("s")


def _lane():
  return lax.iota(jnp.int32, L)


def _bfull(x, dtype=jnp.int32):
  return jnp.full((L,), x, dtype)


def _bcast(vref, idx):
  if isinstance(idx, int) and idx % L == 0:
    row = vref[pl.ds(_c(idx), L)]
    zero = jnp.zeros((L,), row.dtype)
    s = jnp.sum(jnp.where(_lane() == 0, row, zero), dtype=row.dtype)
    return jnp.full((L,), s, row.dtype)
  return plsc.load_gather(vref, [_bfull(idx)])


def _floor_div(p, leafpat):
  q = p / leafpat
  t = q.astype(jnp.int32)
  tf = t.astype(jnp.float32)
  return jnp.where(tf > q, t - 1, t)


def _k1_minmax(n):
  c = n // NW
  wf = 12288
  nwin = (c * 3) // wf

  def body(pts, leaf, mm, pv, lv, rowv):
    wid = _wid()
    lane = _lane()
    pltpu.sync_copy(leaf, lv)
    leafpat = [plsc.load_gather(lv, [(lane + j) % 3]) for j in range(3)]
    acc0 = []
    for _ in range(3):
      acc0 += [_bfull(INT_MAX), _bfull(INT_MIN)]

    def win_loop(w, acc):
      base = wid * _c(c * 3) + w * _c(wf)
      pltpu.sync_copy(pts.at[pl.ds(base, wf)], pv)

      def step(i, acc):
        acc = list(acc)
        for j in range(3):
          p = pv[pl.ds(i * _c(48) + _c(16 * j), L)]
          cc = _floor_div(p, leafpat[j])
          acc[2 * j] = jnp.minimum(acc[2 * j], cc)
          acc[2 * j + 1] = jnp.maximum(acc[2 * j + 1], cc)
        return tuple(acc)

      return _loop(wf // 48, step, acc)

    acc = _loop(nwin, win_loop, tuple(acc0))
    row = _bfull(0)
    for m in range(3):
      vmin = _bfull(INT_MAX)
      vmax = _bfull(INT_MIN)
      for j in range(3):
        cm = (lane + j) % 3 == m
        vmin = jnp.minimum(vmin, jnp.where(cm, acc[2 * j], _bfull(INT_MAX)))
        vmax = jnp.maximum(vmax, jnp.where(cm, acc[2 * j + 1],
                                           _bfull(INT_MIN)))
      smin = jnp.min(vmin)
      smax = jnp.max(vmax)
      row = jnp.where(lane == m, _bfull(smin), row)
      row = jnp.where(lane == m + 3, _bfull(smax), row)
    rowv[...] = row
    pltpu.sync_copy(rowv, mm.at[pl.ds(wid * _c(L), L)])

  return pl.kernel(
      body,
      out_type=jax.ShapeDtypeStruct((NW * L,), jnp.int32),
      mesh=_mesh(),
      compiler_params=pltpu.CompilerParams(needs_layout_passes=False),
      scratch_types=[
          pltpu.VMEM((wf,), jnp.float32),
          pltpu.VMEM((L,), jnp.float32),
          pltpu.VMEM((L,), jnp.int32),
      ],
  )


def _global_minmax(mmv, lane):
  acc = jnp.where(lane < 3, _bfull(INT_MAX), _bfull(INT_MIN))

  def step(w, acc):
    row = mmv[pl.ds(w * _c(L), L)]
    return jnp.where(lane < 3, jnp.minimum(acc, row), jnp.maximum(acc, row))

  return _loop(NW, step, acc)


def _k2_keys_hist(n):
  c = n // NW
  wpt = 4096
  wf = wpt * 3
  nwin = c // wpt

  def body(pts, leaf, mm, keys, hist, pv, lv, mmv, redv, cvec, kv, histv,
           stage):
    wid = _wid()
    lane = _lane()
    pltpu.sync_copy(leaf, lv)
    pltpu.sync_copy(mm, mmv)
    leafpat = [plsc.load_gather(lv, [(lane + j) % 3]) for j in range(3)]
    redv[...] = _global_minmax(mmv, lane)
    mn = [_bcast(redv, m) for m in range(3)]
    mx = [_bcast(redv, m + 3) for m in range(3)]
    d2 = mx[2] - mn[2] + 1
    d1 = mx[1] - mn[1] + 1
    d12 = d1 * d2
    wvec = jnp.where(lane == 0, d12, jnp.where(lane == 1, d2, _bfull(1)))
    mvec = jnp.where(lane == 0, mn[0], jnp.where(lane == 1, mn[1], mn[2]))
    cvec[pl.ds(0, L)] = wvec
    cvec[pl.ds(L, L)] = mvec
    wpat = [plsc.load_gather(cvec, [(lane + j) % 3]) for j in range(3)]
    mpat = [plsc.load_gather(cvec, [L + (lane + j) % 3]) for j in range(3)]

    def zstep(i, _):
      histv[pl.ds(i * _c(L), L)] = _bfull(0)
      return 0

    _loop(RADIX // L, zstep, 0)

    def wstep(win, _):
      base = wid * _c(c) + win * _c(wpt)
      pltpu.sync_copy(pts.at[pl.ds(_c(3) * base, wf)], pv)

      def step(i, _):
        for j in range(3):
          p = pv[pl.ds(i * _c(48) + _c(16 * j), L)]
          cc = _floor_div(p, leafpat[j])
          stage[pl.ds(16 * j, L)] = (cc - mpat[j]) * wpat[j]
        k16 = (plsc.load_gather(stage, [3 * lane]) +
               plsc.load_gather(stage, [3 * lane + 1]) +
               plsc.load_gather(stage, [3 * lane + 2]))
        kv[pl.ds(i * _c(16), L)] = k16
        d16 = k16 & (RADIX - 1)
        cnt, lastm = plsc.scan_count(d16)
        plsc.addupdate_scatter(histv, [d16], cnt, mask=lastm)
        return 0

      _loop(wpt // 16, step, 0)
      pltpu.sync_copy(kv, keys.at[pl.ds(base, wpt)])
      return 0

    _loop(nwin, wstep, 0)
    pltpu.sync_copy(histv, hist.at[pl.ds(wid * _c(RADIX), RADIX)])

  return pl.kernel(
      body,
      out_type=(jax.ShapeDtypeStruct((n,), jnp.int32),
                jax.ShapeDtypeStruct((NW * RADIX,), jnp.int32)),
      mesh=_mesh(),
      compiler_params=pltpu.CompilerParams(needs_layout_passes=False),
      scratch_types=[
          pltpu.VMEM((wf,), jnp.float32),
          pltpu.VMEM((L,), jnp.float32),
          pltpu.VMEM((NW * L,), jnp.int32),
          pltpu.VMEM((L,), jnp.int32),
          pltpu.VMEM((2 * L,), jnp.int32),
          pltpu.VMEM((wpt,), jnp.int32),
          pltpu.VMEM((RADIX,), jnp.int32),
          pltpu.VMEM((48,), jnp.int32),
      ],
  )


def _k_hist(n, shift):
  c = n // NW
  wpt = 8192
  nwin = c // wpt

  def body(keys, hist, kv, histv):
    wid = _wid()
    sh = _bfull(shift)

    def zstep(i, _):
      histv[pl.ds(i * _c(L), L)] = _bfull(0)
      return 0

    _loop(RADIX // L, zstep, 0)

    def wstep(win, _):
      base = wid * _c(c) + win * _c(wpt)
      pltpu.sync_copy(keys.at[pl.ds(base, wpt)], kv)

      def step(i, _):
        k16 = kv[pl.ds(i * _c(16), L)]
        d16 = lax.shift_right_logical(k16, sh) & (RADIX - 1)
        cnt, lastm = plsc.scan_count(d16)
        plsc.addupdate_scatter(histv, [d16], cnt, mask=lastm)
        return 0

      _loop(wpt // 16, step, 0)
      return 0

    _loop(nwin, wstep, 0)
    pltpu.sync_copy(histv, hist.at[pl.ds(wid * _c(RADIX), RADIX)])

  return pl.kernel(
      body,
      out_type=jax.ShapeDtypeStruct((NW * RADIX,), jnp.int32),
      mesh=_mesh(),
      compiler_params=pltpu.CompilerParams(needs_layout_passes=False),
      scratch_types=[
          pltpu.VMEM((wpt,), jnp.int32),
          pltpu.VMEM((RADIX,), jnp.int32),
      ],
  )


def _base_offsets(histm, basev, wid, lane):
  wid_b = _bfull(wid)

  def dstep(dc, carry):
    tot = _bfull(0)
    below = _bfull(0)
    for w in range(NW):
      row = histm[pl.ds(dc * _c(L) + _c(w * RADIX), L)]
      tot = tot + row
      below = below + jnp.where(_bfull(w) < wid_b, row, _bfull(0))
    incl = plsc.cumsum(tot)
    basev[pl.ds(dc * _c(L), L)] = incl - tot + carry + below
    return carry + _bfull(jnp.sum(tot, dtype=jnp.int32))

  _loop(RADIX // L, dstep, _bfull(0))


def _k_permute(n, shift, first):
  c = n // NW
  wpt = 4096
  wf = wpt * 3
  nwin = c // wpt
  f32 = jnp.float32

  def body(keys, hist, *rest):
    if first:
      (pts, keys_o, x_o, y_o, z_o,
       histm, basev, kv, xv, yv, zv, posv, pv, stage, sem) = rest
    else:
      (x_i, y_i, z_i, keys_o, x_o, y_o, z_o,
       histm, basev, kv, xv, yv, zv, posv, sem) = rest
    wid = _wid()
    lane = _lane()
    sh = _bfull(shift)
    pltpu.sync_copy(hist, histm)
    _base_offsets(histm, basev, wid, lane)

    def wstep(win, _):
      base = wid * _c(c) + win * _c(wpt)
      pltpu.sync_copy(keys.at[pl.ds(base, wpt)], kv)
      if first:
        pltpu.sync_copy(pts.at[pl.ds(_c(3) * base, wf)], pv)

        def tstep(i, _):
          stage[pl.ds(0, L)] = pv[pl.ds(i * _c(48), L)]
          stage[pl.ds(L, L)] = pv[pl.ds(i * _c(48) + _c(16), L)]
          stage[pl.ds(2 * L, L)] = pv[pl.ds(i * _c(48) + _c(32), L)]
          xv[pl.ds(i * _c(16), L)] = plsc.load_gather(stage, [3 * lane])
          yv[pl.ds(i * _c(16), L)] = plsc.load_gather(stage, [3 * lane + 1])
          zv[pl.ds(i * _c(16), L)] = plsc.load_gather(stage, [3 * lane + 2])
          return 0

        _loop(wpt // 16, tstep, 0)
      else:
        pltpu.sync_copy(x_i.at[pl.ds(base, wpt)], xv)
        pltpu.sync_copy(y_i.at[pl.ds(base, wpt)], yv)
        pltpu.sync_copy(z_i.at[pl.ds(base, wpt)], zv)

      def step(i, _):
        k16 = kv[pl.ds(i * _c(16), L)]
        d16 = lax.shift_right_logical(k16, sh) & (RADIX - 1)
        cnt, lastm = plsc.scan_count(d16)
        cur = plsc.load_gather(basev, [d16])
        posv[pl.ds(i * _c(16), L)] = cur + cnt - 1
        plsc.addupdate_scatter(basev, [d16], cnt, mask=lastm)
        return 0

      _loop(wpt // 16, step, 0)
      cps = [pltpu.async_copy(kv, keys_o.at[posv], sem),
             pltpu.async_copy(xv, x_o.at[posv], sem),
             pltpu.async_copy(yv, y_o.at[posv], sem),
             pltpu.async_copy(zv, z_o.at[posv], sem)]
      for cp in cps:
        cp.wait()
      return 0

    _loop(nwin, wstep, 0)

  scratch = [
      pltpu.VMEM((NW * RADIX,), jnp.int32),
      pltpu.VMEM((RADIX,), jnp.int32),
      pltpu.VMEM((wpt,), jnp.int32),
      pltpu.VMEM((wpt,), f32),
      pltpu.VMEM((wpt,), f32),
      pltpu.VMEM((wpt,), f32),
      pltpu.VMEM((wpt,), jnp.int32),
  ]
  if first:
    scratch += [pltpu.VMEM((wf,), f32), pltpu.VMEM((48,), f32)]
  scratch += [pltpu.SemaphoreType.DMA]
  return pl.kernel(
      body,
      out_type=(jax.ShapeDtypeStruct((n,), jnp.int32),
                jax.ShapeDtypeStruct((n,), f32),
                jax.ShapeDtypeStruct((n,), f32),
                jax.ShapeDtypeStruct((n,), f32)),
      mesh=_mesh(),
      compiler_params=pltpu.CompilerParams(needs_layout_passes=False),
      scratch_types=scratch,
  )


def _k8a_bounds(n):
  c = n // NW
  wpt = 8192
  nwin = c // wpt

  def body(keys, bc, kpad, prevv, rowv):
    wid = _wid()
    lane = _lane()
    prevv[...] = _bfull(-1)

    @pl.when(wid > 0)
    def _():
      pltpu.sync_copy(keys.at[pl.ds(wid * _c(c) - _c(L), L)], prevv)

    prev_b = _bcast(prevv, L - 1)
    plsc.store_scatter(kpad, [_bfull(15)], prev_b, mask=lane == 0)
    def wstep(win, acc):
      base = wid * _c(c) + win * _c(wpt)
      pltpu.sync_copy(keys.at[pl.ds(base, wpt)], kpad.at[pl.ds(L, wpt)])

      def step(i, acc):
        k16 = kpad[pl.ds(i * _c(16) + _c(L), L)]
        p16 = plsc.load_gather(kpad, [i * _c(16) + _c(15) + lane])
        return acc + jnp.where(k16 != p16, _bfull(1), _bfull(0))

      acc = _loop(wpt // 16, step, acc)
      lastk = _bcast(kpad, L + wpt - 1)
      plsc.store_scatter(kpad, [_bfull(15)], lastk, mask=lane == 0)
      return acc

    acc = _loop(nwin, wstep, _bfull(0))
    rowv[...] = jnp.where(lane == 0, _bfull(jnp.sum(acc, dtype=jnp.int32)),
                          _bfull(0))
    pltpu.sync_copy(rowv, bc.at[pl.ds(wid * _c(L), L)])

  return pl.kernel(
      body,
      out_type=jax.ShapeDtypeStruct((NW * L,), jnp.int32),
      mesh=_mesh(),
      compiler_params=pltpu.CompilerParams(needs_layout_passes=False),
      scratch_types=[
          pltpu.VMEM((wpt + L,), jnp.int32),
          pltpu.VMEM((L,), jnp.int32),
          pltpu.VMEM((L,), jnp.int32),
      ],
  )


def _k8b_segsum(n):
  c = n // NW
  wpt = 4096
  nwin = c // wpt
  nsum = n + 2 * wpt
  f32 = jnp.float32

  def body(keys, xs, ys, zs, bc, sx_o, sy_o, sz_o, sc_o, meta,
           kpad, xv, yv, zv, tx, ty, tz, tc, ixv, bcv, sxv, metav, tmpv,
           sem):
    wid = _wid()
    lane = _lane()
    pltpu.sync_copy(bc, bcv)
    wid_b = _bfull(wid)
    gbase = _bfull(0)
    for w in range(NW):
      cb = _bcast(bcv, L * w)
      gbase = gbase + jnp.where(_bfull(w) < wid_b, cb, _bfull(0))

    pbase = pl.multiple_of(jnp.maximum(wid * _c(c) - _c(L), _c(0)), L)
    pltpu.sync_copy(keys.at[pl.ds(pbase, L)], tmpv)
    is_first = jnp.broadcast_to(wid == 0, (L,))
    prev_b = jnp.where(is_first, _bfull(-1), _bcast(tmpv, L - 1))
    plsc.store_scatter(kpad, [_bfull(15)], prev_b, mask=lane == 0)
    sxv[pl.ds(0, L)] = jnp.zeros((L,), f32)
    sxv[pl.ds(L, L)] = jnp.zeros((L,), f32)

    zero_f = jnp.zeros((L,), f32)
    gcur = gbase
    opengid = gbase - 1
    ocx = zero_f
    ocy = zero_f
    ocz = zero_f
    occ = zero_f
    anyend = _bfull(0)

    def wstep(win, wcarry):
      gcur, opengid, ocx, ocy, ocz, occ, anyend = wcarry
      base = wid * _c(c) + win * _c(wpt)
      pltpu.sync_copy(keys.at[pl.ds(base, wpt)], kpad.at[pl.ds(L, wpt)])
      pltpu.sync_copy(xs.at[pl.ds(base, wpt)], xv)
      pltpu.sync_copy(ys.at[pl.ds(base, wpt)], yv)
      pltpu.sync_copy(zs.at[pl.ds(base, wpt)], zv)
      nbase = pl.multiple_of(jnp.minimum(base + _c(wpt), _c(n - L)), L)
      pltpu.sync_copy(keys.at[pl.ds(nbase, L)], tmpv)
      is_last = jnp.broadcast_to(base + _c(wpt) >= _c(n), (L,))
      nxt_b = jnp.where(is_last, _bfull(-2), _bcast(tmpv, 0))
      plsc.store_scatter(kpad, [_bfull(L + wpt)], nxt_b, mask=lane == 0)

      carry0 = (gcur, opengid, ocx, ocy, ocz, occ, anyend)

      def step(i, carry):
        gcur, opengid, ocx, ocy, ocz, occ, anyend = carry
        k16 = kpad[pl.ds(i * _c(16) + _c(L), L)]
        p16 = plsc.load_gather(kpad, [i * _c(16) + _c(15) + lane])
        n16 = plsc.load_gather(kpad, [i * _c(16) + _c(17) + lane])
        nb = k16 != p16
        ge = k16 != n16
        nbi = jnp.where(nb, _bfull(1), _bfull(0))
        incl = plsc.cumsum(nbi)
        gid16 = gcur + incl - 1
        cnt, _lm = plsc.scan_count(gid16)
        firstg = gid16 == opengid
        lge = _bfull(jnp.max(jnp.where(ge, lane, _bfull(-1))))
        nge = jnp.sum(jnp.where(ge, _bfull(1), _bfull(0)), dtype=jnp.int32)
        hasge = _bfull(nge) > 0
        outs = []
        for rres, vals, oc in ((tx, xv, ocx), (ty, yv, ocy), (tz, zv, ocz),
                               (tc, None, occ)):
          if vals is None:
            tot16 = cnt.astype(f32) + jnp.where(firstg, oc, zero_f)
            sfx = (_bfull(15) - lge).astype(f32)
          else:
            v16 = vals[pl.ds(i * _c(16), L)]
            s16 = plsc.cumsum(v16)
            sxv[pl.ds(8, L)] = s16
            sstart = plsc.load_gather(sxv, [8 + lane - cnt])
            tot16 = s16 - sstart + jnp.where(firstg, oc, zero_f)
            s15 = _bfull(jnp.sum(v16), f32)
            sfx = s15 - plsc.load_gather(sxv, [8 + lge])
          rres[pl.ds(i * _c(16), L)] = tot16
          outs.append(jnp.where(hasge, sfx, oc + sfx))
        ocx, ocy, ocz, occ = outs
        pos16 = i * _c(16) + lane
        ixv[pl.ds(i * _c(16), L)] = jnp.where(ge, gid16, _bfull(n) + pos16)
        anyend = anyend | jnp.where(hasge, _bfull(1), _bfull(0))
        gcur = gcur + _bfull(jnp.sum(nbi, dtype=jnp.int32))
        opengid = gcur - 1
        return (gcur, opengid, ocx, ocy, ocz, occ, anyend)

      (gcur, opengid, ocx, ocy, ocz, occ, anyend) = (
          _loop(wpt // 16, step, carry0))
      cps = [pltpu.async_copy(tx, sx_o.at[ixv], sem),
             pltpu.async_copy(ty, sy_o.at[ixv], sem),
             pltpu.async_copy(tz, sz_o.at[ixv], sem),
             pltpu.async_copy(tc, sc_o.at[ixv], sem)]
      for cp in cps:
        cp.wait()
      lastk = _bcast(kpad, L + wpt - 1)
      plsc.store_scatter(kpad, [_bfull(15)], lastk, mask=lane == 0)
      return (gcur, opengid, ocx, ocy, ocz, occ, anyend)

    (gcur, opengid, ocx, ocy, ocz, occ, anyend) = _loop(
        nwin, wstep, (gcur, opengid, ocx, ocy, ocz, occ, anyend))

    metarow = jnp.where(lane == 0, ocx,
               jnp.where(lane == 1, ocy,
                jnp.where(lane == 2, ocz,
                 jnp.where(lane == 3, occ,
                  jnp.where(lane == 4, anyend.astype(f32), zero_f)))))
    metav[...] = metarow
    pltpu.sync_copy(metav, meta.at[pl.ds(wid * _c(L), L)])

  sde = jax.ShapeDtypeStruct((nsum,), f32)
  return pl.kernel(
      body,
      out_type=(sde, sde, sde, sde,
                jax.ShapeDtypeStruct((NW * L,), f32)),
      mesh=_mesh(),
      compiler_params=pltpu.CompilerParams(needs_layout_passes=False),
      scratch_types=[
          pltpu.VMEM((wpt + 2 * L,), jnp.int32),
          pltpu.VMEM((wpt,), f32),
          pltpu.VMEM((wpt,), f32),
          pltpu.VMEM((wpt,), f32),
          pltpu.VMEM((wpt,), f32),
          pltpu.VMEM((wpt,), f32),
          pltpu.VMEM((wpt,), f32),
          pltpu.VMEM((wpt,), f32),
          pltpu.VMEM((wpt,), jnp.int32),
          pltpu.VMEM((NW * L,), jnp.int32),
          pltpu.VMEM((2 * L,), f32),
          pltpu.VMEM((L,), f32),
          pltpu.VMEM((L,), jnp.int32),
          pltpu.SemaphoreType.DMA,
      ],
  )


def _k9_means(n, nsum):
  c = n // NW
  wrow = 2048
  nwin = c // wrow
  f32 = jnp.float32

  def body(sx, sy, sz, sc, bc, meta, out, xwv, ywv, zwv, cwv, outv, bcv,
           tmv, cgv, cvv):
    wid = _wid()
    lane = _lane()
    pltpu.sync_copy(bc, bcv)
    pltpu.sync_copy(meta, tmv)
    zero_f = jnp.zeros((L,), f32)
    ng = _bfull(0)
    for w in range(NW):
      ng = ng + _bcast(bcv, L * w)
    carry = zero_f
    gb = _bfull(0)
    for w in range(NW):
      m16 = tmv[pl.ds(L * w, L)]
      trail = jnp.where(lane < 4, m16, zero_f)
      he = plsc.load_gather(tmv, [_bfull(L * w + 4)]) > 0.5
      tg = gb - 1
      cgv[pl.ds(L * w, L)] = jnp.where(he & (tg >= 0), tg, _bfull(nsum - L))
      cvv[pl.ds(L * w, L)] = jnp.where(he, carry, zero_f)
      carry = jnp.where(he, trail, carry + trail)
      gb = gb + _bcast(bcv, L * w)

    lane0 = lane == 0

    def wstep(win, _):
      rowbase = wid * _c(c) + win * _c(wrow)
      pltpu.sync_copy(sx.at[pl.ds(rowbase, wrow)], xwv)
      pltpu.sync_copy(sy.at[pl.ds(rowbase, wrow)], ywv)
      pltpu.sync_copy(sz.at[pl.ds(rowbase, wrow)], zwv)
      pltpu.sync_copy(sc.at[pl.ds(rowbase, wrow)], cwv)
      rb = _bfull(rowbase)
      for w in range(NW):
        tg = _bcast(cgv, L * w)
        rel = tg - rb
        inwin = (rel >= 0) & (rel < wrow) & lane0
        relc = jnp.maximum(rel, 0)
        plsc.addupdate_scatter(xwv, [relc], _bcast(cvv, L * w), mask=inwin)
        plsc.addupdate_scatter(ywv, [relc], _bcast(cvv, L * w + 1),
                               mask=inwin)
        plsc.addupdate_scatter(zwv, [relc], _bcast(cvv, L * w + 2),
                               mask=inwin)
        plsc.addupdate_scatter(cwv, [relc], _bcast(cvv, L * w + 3),
                               mask=inwin)

      def ostep(i, _):
        for j in range(3):
          a = 16 * j + lane
          relrow = _bfull(i * _c(16)) + a // 3
          comp = a % 3
          gx = plsc.load_gather(xwv, [relrow])
          gy = plsc.load_gather(ywv, [relrow])
          gz = plsc.load_gather(zwv, [relrow])
          val = jnp.where(comp == 0, gx, jnp.where(comp == 1, gy, gz))
          cntv = plsc.load_gather(cwv, [relrow])
          valid = (rb + relrow) < ng
          outv[pl.ds(i * _c(48) + _c(16 * j), L)] = jnp.where(
              valid, val / cntv, zero_f)
        return 0

      _loop(wrow // 16, ostep, 0)
      pltpu.sync_copy(outv, out.at[pl.ds(_c(3) * rowbase, 3 * wrow)])
      return 0

    _loop(nwin, wstep, 0)

  return pl.kernel(
      body,
      out_type=jax.ShapeDtypeStruct((3 * n,), f32),
      mesh=_mesh(),
      compiler_params=pltpu.CompilerParams(needs_layout_passes=False),
      scratch_types=[
          pltpu.VMEM((wrow,), f32),
          pltpu.VMEM((wrow,), f32),
          pltpu.VMEM((wrow,), f32),
          pltpu.VMEM((wrow,), f32),
          pltpu.VMEM((3 * wrow,), f32),
          pltpu.VMEM((NW * L,), jnp.int32),
          pltpu.VMEM((NW * L,), f32),
          pltpu.VMEM((NW * L,), jnp.int32),
          pltpu.VMEM((NW * L,), f32),
      ],
  )


def kernel(points, leaf_size):
  n = points.shape[0]
  pts_flat = points.reshape(-1)
  leaf16 = jnp.concatenate(
      [leaf_size.astype(jnp.float32),
       jnp.ones((13,), jnp.float32)])

  mm = _k1_minmax(n)(pts_flat, leaf16)
  keys, hist0 = _k2_keys_hist(n)(pts_flat, leaf16, mm)
  k1s, x1, y1, z1 = _k_permute(n, SHIFTS[0], True)(keys, hist0, pts_flat)
  h1 = _k_hist(n, SHIFTS[1])(k1s)
  k2s, x2, y2, z2 = _k_permute(n, SHIFTS[1], False)(k1s, h1, x1, y1, z1)
  h2 = _k_hist(n, SHIFTS[2])(k2s)
  k3s, x3, y3, z3 = _k_permute(n, SHIFTS[2], False)(k2s, h2, x2, y2, z2)
  bc = _k8a_bounds(n)(k3s)
  s_x, s_y, s_z, s_c, meta = _k8b_segsum(n)(k3s, x3, y3, z3, bc)
  nsum = s_x.shape[0]
  outflat = _k9_means(n, nsum)(s_x, s_y, s_z, s_c, bc, meta)
  ngroups = jnp.sum(bc.reshape(NW, L)[:, 0])
  out = outflat.reshape(n, 3)
  mask = jnp.arange(n) < ngroups
  return (out, mask)


# speedup vs baseline: 13.2906x; 1.0115x over previous
import functools

import jax
import jax.numpy as jnp
from jax import lax
from jax.experimental import pallas as pl
from jax.experimental.pallas import tpu as pltpu
from jax.experimental.pallas import tpu_sc as plsc

NC = 2
NS = 16
NW = NC * NS
L = 16
NBITS = 11
RADIX = 1 << NBITS
SHIFTS = (0, NBITS, 2 * NBITS)
INT_MAX = 2**31 - 1
INT_MIN = -(2**31)


def _c(x):
  return jnp.int32(x)


def _loop(n, body, carry):
  def wrap(_, c):
    i, inner = c
    return (i + _c(1), body(i, inner))

  return lax.fori_loop(0, n, wrap, (_c(0), carry))[1]


def _mesh():
  return plsc.VectorSubcoreMesh(core_axis_name="c", subcore_axis_name="s")


def _wid():
  return lax.axis_index("c") * NS + lax.axis_index("s")


def _lane():
  return lax.iota(jnp.int32, L)


def _bfull(x, dtype=jnp.int32):
  return jnp.full((L,), x, dtype)


def _bcast(vref, idx):
  if isinstance(idx, int) and idx % L == 0:
    row = vref[pl.ds(_c(idx), L)]
    zero = jnp.zeros((L,), row.dtype)
    s = jnp.sum(jnp.where(_lane() == 0, row, zero), dtype=row.dtype)
    return jnp.full((L,), s, row.dtype)
  return plsc.load_gather(vref, [_bfull(idx)])


def _floor_div(p, leafpat):
  q = p / leafpat
  t = q.astype(jnp.int32)
  tf = t.astype(jnp.float32)
  return jnp.where(tf > q, t - 1, t)


def _k1_minmax(n):
  c = n // NW
  wf = 12288
  nwin = (c * 3) // wf

  def body(pts, leaf, mm, pv, lv, rowv):
    wid = _wid()
    lane = _lane()
    pltpu.sync_copy(leaf, lv)
    leafpat = [plsc.load_gather(lv, [(lane + j) % 3]) for j in range(3)]
    acc0 = []
    for _ in range(3):
      acc0 += [_bfull(INT_MAX), _bfull(INT_MIN)]

    def win_loop(w, acc):
      base = wid * _c(c * 3) + w * _c(wf)
      pltpu.sync_copy(pts.at[pl.ds(base, wf)], pv)

      def step(i, acc):
        acc = list(acc)
        for j in range(3):
          p = pv[pl.ds(i * _c(48) + _c(16 * j), L)]
          cc = _floor_div(p, leafpat[j])
          acc[2 * j] = jnp.minimum(acc[2 * j], cc)
          acc[2 * j + 1] = jnp.maximum(acc[2 * j + 1], cc)
        return tuple(acc)

      return _loop(wf // 48, step, acc)

    acc = _loop(nwin, win_loop, tuple(acc0))
    row = _bfull(0)
    for m in range(3):
      vmin = _bfull(INT_MAX)
      vmax = _bfull(INT_MIN)
      for j in range(3):
        cm = (lane + j) % 3 == m
        vmin = jnp.minimum(vmin, jnp.where(cm, acc[2 * j], _bfull(INT_MAX)))
        vmax = jnp.maximum(vmax, jnp.where(cm, acc[2 * j + 1],
                                           _bfull(INT_MIN)))
      smin = jnp.min(vmin)
      smax = jnp.max(vmax)
      row = jnp.where(lane == m, _bfull(smin), row)
      row = jnp.where(lane == m + 3, _bfull(smax), row)
    rowv[...] = row
    pltpu.sync_copy(rowv, mm.at[pl.ds(wid * _c(L), L)])

  return pl.kernel(
      body,
      out_type=jax.ShapeDtypeStruct((NW * L,), jnp.int32),
      mesh=_mesh(),
      compiler_params=pltpu.CompilerParams(needs_layout_passes=False),
      scratch_types=[
          pltpu.VMEM((wf,), jnp.float32),
          pltpu.VMEM((L,), jnp.float32),
          pltpu.VMEM((L,), jnp.int32),
      ],
  )


def _global_minmax(mmv, lane):
  acc = jnp.where(lane < 3, _bfull(INT_MAX), _bfull(INT_MIN))

  def step(w, acc):
    row = mmv[pl.ds(w * _c(L), L)]
    return jnp.where(lane < 3, jnp.minimum(acc, row), jnp.maximum(acc, row))

  return _loop(NW, step, acc)


def _k2_keys_hist(n):
  c = n // NW
  wpt = 4096
  wf = wpt * 3
  nwin = c // wpt

  def body(pts, leaf, mm, keys, hist, pv, lv, mmv, redv, cvec, kv, histv,
           stage):
    wid = _wid()
    lane = _lane()
    pltpu.sync_copy(leaf, lv)
    pltpu.sync_copy(mm, mmv)
    leafpat = [plsc.load_gather(lv, [(lane + j) % 3]) for j in range(3)]
    redv[...] = _global_minmax(mmv, lane)
    mn = [_bcast(redv, m) for m in range(3)]
    mx = [_bcast(redv, m + 3) for m in range(3)]
    d2 = mx[2] - mn[2] + 1
    d1 = mx[1] - mn[1] + 1
    d12 = d1 * d2
    wvec = jnp.where(lane == 0, d12, jnp.where(lane == 1, d2, _bfull(1)))
    mvec = jnp.where(lane == 0, mn[0], jnp.where(lane == 1, mn[1], mn[2]))
    cvec[pl.ds(0, L)] = wvec
    cvec[pl.ds(L, L)] = mvec
    wpat = [plsc.load_gather(cvec, [(lane + j) % 3]) for j in range(3)]
    mpat = [plsc.load_gather(cvec, [L + (lane + j) % 3]) for j in range(3)]

    def zstep(i, _):
      histv[pl.ds(i * _c(L), L)] = _bfull(0)
      return 0

    _loop(RADIX // L, zstep, 0)

    def wstep(win, _):
      base = wid * _c(c) + win * _c(wpt)
      pltpu.sync_copy(pts.at[pl.ds(_c(3) * base, wf)], pv)

      def step(i, _):
        for j in range(3):
          p = pv[pl.ds(i * _c(48) + _c(16 * j), L)]
          cc = _floor_div(p, leafpat[j])
          stage[pl.ds(16 * j, L)] = (cc - mpat[j]) * wpat[j]
        k16 = (plsc.load_gather(stage, [3 * lane]) +
               plsc.load_gather(stage, [3 * lane + 1]) +
               plsc.load_gather(stage, [3 * lane + 2]))
        kv[pl.ds(i * _c(16), L)] = k16
        d16 = k16 & (RADIX - 1)
        cnt, lastm = plsc.scan_count(d16)
        plsc.addupdate_scatter(histv, [d16], cnt, mask=lastm)
        return 0

      _loop(wpt // 16, step, 0)
      pltpu.sync_copy(kv, keys.at[pl.ds(base, wpt)])
      return 0

    _loop(nwin, wstep, 0)
    pltpu.sync_copy(histv, hist.at[pl.ds(wid * _c(RADIX), RADIX)])

  return pl.kernel(
      body,
      out_type=(jax.ShapeDtypeStruct((n,), jnp.int32),
                jax.ShapeDtypeStruct((NW * RADIX,), jnp.int32)),
      mesh=_mesh(),
      compiler_params=pltpu.CompilerParams(needs_layout_passes=False),
      scratch_types=[
          pltpu.VMEM((wf,), jnp.float32),
          pltpu.VMEM((L,), jnp.float32),
          pltpu.VMEM((NW * L,), jnp.int32),
          pltpu.VMEM((L,), jnp.int32),
          pltpu.VMEM((2 * L,), jnp.int32),
          pltpu.VMEM((wpt,), jnp.int32),
          pltpu.VMEM((RADIX,), jnp.int32),
          pltpu.VMEM((48,), jnp.int32),
      ],
  )


def _k_hist(n, shift):
  c = n // NW
  wpt = 8192
  nwin = c // wpt

  def body(keys, hist, kv, histv):
    wid = _wid()
    sh = _bfull(shift)

    def zstep(i, _):
      histv[pl.ds(i * _c(L), L)] = _bfull(0)
      return 0

    _loop(RADIX // L, zstep, 0)

    def wstep(win, _):
      base = wid * _c(c) + win * _c(wpt)
      pltpu.sync_copy(keys.at[pl.ds(base, wpt)], kv)

      def step(i, _):
        k16 = kv[pl.ds(i * _c(16), L)]
        d16 = lax.shift_right_logical(k16, sh) & (RADIX - 1)
        cnt, lastm = plsc.scan_count(d16)
        plsc.addupdate_scatter(histv, [d16], cnt, mask=lastm)
        return 0

      _loop(wpt // 16, step, 0)
      return 0

    _loop(nwin, wstep, 0)
    pltpu.sync_copy(histv, hist.at[pl.ds(wid * _c(RADIX), RADIX)])

  return pl.kernel(
      body,
      out_type=jax.ShapeDtypeStruct((NW * RADIX,), jnp.int32),
      mesh=_mesh(),
      compiler_params=pltpu.CompilerParams(needs_layout_passes=False),
      scratch_types=[
          pltpu.VMEM((wpt,), jnp.int32),
          pltpu.VMEM((RADIX,), jnp.int32),
      ],
  )


def _base_offsets(histm, basev, wid, lane):
  wid_b = _bfull(wid)

  def dstep(dc, carry):
    tot = _bfull(0)
    below = _bfull(0)
    for w in range(NW):
      row = histm[pl.ds(dc * _c(L) + _c(w * RADIX), L)]
      tot = tot + row
      below = below + jnp.where(_bfull(w) < wid_b, row, _bfull(0))
    incl = plsc.cumsum(tot)
    basev[pl.ds(dc * _c(L), L)] = incl - tot + carry + below
    return carry + _bfull(jnp.sum(tot, dtype=jnp.int32))

  _loop(RADIX // L, dstep, _bfull(0))


def _k_permute(n, shift, first):
  c = n // NW
  wpt = 4096 if first else 8192
  wf = wpt * 3
  nwin = c // wpt
  f32 = jnp.float32

  def body(keys, hist, *rest):
    if first:
      (pts, keys_o, x_o, y_o, z_o,
       histm, basev, kv, xv, yv, zv, posv, pv, stage, sem) = rest
    else:
      (x_i, y_i, z_i, keys_o, x_o, y_o, z_o,
       histm, basev, kv, xv, yv, zv, posv, sem) = rest
    wid = _wid()
    lane = _lane()
    sh = _bfull(shift)
    pltpu.sync_copy(hist, histm)
    _base_offsets(histm, basev, wid, lane)

    def wstep(win, _):
      base = wid * _c(c) + win * _c(wpt)
      pltpu.sync_copy(keys.at[pl.ds(base, wpt)], kv)
      if first:
        pltpu.sync_copy(pts.at[pl.ds(_c(3) * base, wf)], pv)

        def tstep(i, _):
          stage[pl.ds(0, L)] = pv[pl.ds(i * _c(48), L)]
          stage[pl.ds(L, L)] = pv[pl.ds(i * _c(48) + _c(16), L)]
          stage[pl.ds(2 * L, L)] = pv[pl.ds(i * _c(48) + _c(32), L)]
          xv[pl.ds(i * _c(16), L)] = plsc.load_gather(stage, [3 * lane])
          yv[pl.ds(i * _c(16), L)] = plsc.load_gather(stage, [3 * lane + 1])
          zv[pl.ds(i * _c(16), L)] = plsc.load_gather(stage, [3 * lane + 2])
          return 0

        _loop(wpt // 16, tstep, 0)
      else:
        pltpu.sync_copy(x_i.at[pl.ds(base, wpt)], xv)
        pltpu.sync_copy(y_i.at[pl.ds(base, wpt)], yv)
        pltpu.sync_copy(z_i.at[pl.ds(base, wpt)], zv)

      def step(i, _):
        k16 = kv[pl.ds(i * _c(16), L)]
        d16 = lax.shift_right_logical(k16, sh) & (RADIX - 1)
        cnt, lastm = plsc.scan_count(d16)
        cur = plsc.load_gather(basev, [d16])
        posv[pl.ds(i * _c(16), L)] = cur + cnt - 1
        plsc.addupdate_scatter(basev, [d16], cnt, mask=lastm)
        return 0

      _loop(wpt // 16, step, 0)
      cps = [pltpu.async_copy(kv, keys_o.at[posv], sem),
             pltpu.async_copy(xv, x_o.at[posv], sem),
             pltpu.async_copy(yv, y_o.at[posv], sem),
             pltpu.async_copy(zv, z_o.at[posv], sem)]
      for cp in cps:
        cp.wait()
      return 0

    _loop(nwin, wstep, 0)

  scratch = [
      pltpu.VMEM((NW * RADIX,), jnp.int32),
      pltpu.VMEM((RADIX,), jnp.int32),
      pltpu.VMEM((wpt,), jnp.int32),
      pltpu.VMEM((wpt,), f32),
      pltpu.VMEM((wpt,), f32),
      pltpu.VMEM((wpt,), f32),
      pltpu.VMEM((wpt,), jnp.int32),
  ]
  if first:
    scratch += [pltpu.VMEM((wf,), f32), pltpu.VMEM((48,), f32)]
  scratch += [pltpu.SemaphoreType.DMA]
  return pl.kernel(
      body,
      out_type=(jax.ShapeDtypeStruct((n,), jnp.int32),
                jax.ShapeDtypeStruct((n,), f32),
                jax.ShapeDtypeStruct((n,), f32),
                jax.ShapeDtypeStruct((n,), f32)),
      mesh=_mesh(),
      compiler_params=pltpu.CompilerParams(needs_layout_passes=False),
      scratch_types=scratch,
  )


def _k8a_bounds(n):
  c = n // NW
  wpt = 8192
  nwin = c // wpt

  def body(keys, bc, kpad, prevv, rowv):
    wid = _wid()
    lane = _lane()
    prevv[...] = _bfull(-1)

    @pl.when(wid > 0)
    def _():
      pltpu.sync_copy(keys.at[pl.ds(wid * _c(c) - _c(L), L)], prevv)

    prev_b = _bcast(prevv, L - 1)
    plsc.store_scatter(kpad, [_bfull(15)], prev_b, mask=lane == 0)
    def wstep(win, acc):
      base = wid * _c(c) + win * _c(wpt)
      pltpu.sync_copy(keys.at[pl.ds(base, wpt)], kpad.at[pl.ds(L, wpt)])

      def step(i, acc):
        k16 = kpad[pl.ds(i * _c(16) + _c(L), L)]
        p16 = plsc.load_gather(kpad, [i * _c(16) + _c(15) + lane])
        return acc + jnp.where(k16 != p16, _bfull(1), _bfull(0))

      acc = _loop(wpt // 16, step, acc)
      lastk = _bcast(kpad, L + wpt - 1)
      plsc.store_scatter(kpad, [_bfull(15)], lastk, mask=lane == 0)
      return acc

    acc = _loop(nwin, wstep, _bfull(0))
    rowv[...] = jnp.where(lane == 0, _bfull(jnp.sum(acc, dtype=jnp.int32)),
                          _bfull(0))
    pltpu.sync_copy(rowv, bc.at[pl.ds(wid * _c(L), L)])

  return pl.kernel(
      body,
      out_type=jax.ShapeDtypeStruct((NW * L,), jnp.int32),
      mesh=_mesh(),
      compiler_params=pltpu.CompilerParams(needs_layout_passes=False),
      scratch_types=[
          pltpu.VMEM((wpt + L,), jnp.int32),
          pltpu.VMEM((L,), jnp.int32),
          pltpu.VMEM((L,), jnp.int32),
      ],
  )


def _k8b_segsum(n):
  c = n // NW
  wpt = 8192
  nwin = c // wpt
  nsum = n + 2 * wpt
  f32 = jnp.float32

  def body(keys, xs, ys, zs, bc, sx_o, sy_o, sz_o, sc_o, meta,
           kpad, xv, yv, zv, tx, ty, tz, tc, ixv, bcv, sxv, metav, tmpv,
           sem):
    wid = _wid()
    lane = _lane()
    pltpu.sync_copy(bc, bcv)
    wid_b = _bfull(wid)
    gbase = _bfull(0)
    for w in range(NW):
      cb = _bcast(bcv, L * w)
      gbase = gbase + jnp.where(_bfull(w) < wid_b, cb, _bfull(0))

    pbase = pl.multiple_of(jnp.maximum(wid * _c(c) - _c(L), _c(0)), L)
    pltpu.sync_copy(keys.at[pl.ds(pbase, L)], tmpv)
    is_first = jnp.broadcast_to(wid == 0, (L,))
    prev_b = jnp.where(is_first, _bfull(-1), _bcast(tmpv, L - 1))
    plsc.store_scatter(kpad, [_bfull(15)], prev_b, mask=lane == 0)
    sxv[pl.ds(0, L)] = jnp.zeros((L,), f32)
    sxv[pl.ds(L, L)] = jnp.zeros((L,), f32)

    zero_f = jnp.zeros((L,), f32)
    gcur = gbase
    opengid = gbase - 1
    ocx = zero_f
    ocy = zero_f
    ocz = zero_f
    occ = zero_f
    anyend = _bfull(0)

    def wstep(win, wcarry):
      gcur, opengid, ocx, ocy, ocz, occ, anyend = wcarry
      base = wid * _c(c) + win * _c(wpt)
      pltpu.sync_copy(keys.at[pl.ds(base, wpt)], kpad.at[pl.ds(L, wpt)])
      pltpu.sync_copy(xs.at[pl.ds(base, wpt)], xv)
      pltpu.sync_copy(ys.at[pl.ds(base, wpt)], yv)
      pltpu.sync_copy(zs.at[pl.ds(base, wpt)], zv)
      nbase = pl.multiple_of(jnp.minimum(base + _c(wpt), _c(n - L)), L)
      pltpu.sync_copy(keys.at[pl.ds(nbase, L)], tmpv)
      is_last = jnp.broadcast_to(base + _c(wpt) >= _c(n), (L,))
      nxt_b = jnp.where(is_last, _bfull(-2), _bcast(tmpv, 0))
      plsc.store_scatter(kpad, [_bfull(L + wpt)], nxt_b, mask=lane == 0)

      carry0 = (gcur, opengid, ocx, ocy, ocz, occ, anyend)

      def step(i, carry):
        gcur, opengid, ocx, ocy, ocz, occ, anyend = carry
        k16 = kpad[pl.ds(i * _c(16) + _c(L), L)]
        p16 = plsc.load_gather(kpad, [i * _c(16) + _c(15) + lane])
        n16 = plsc.load_gather(kpad, [i * _c(16) + _c(17) + lane])
        nb = k16 != p16
        ge = k16 != n16
        nbi = jnp.where(nb, _bfull(1), _bfull(0))
        incl = plsc.cumsum(nbi)
        gid16 = gcur + incl - 1
        cnt, _lm = plsc.scan_count(gid16)
        firstg = gid16 == opengid
        lge = _bfull(jnp.max(jnp.where(ge, lane, _bfull(-1))))
        nge = jnp.sum(jnp.where(ge, _bfull(1), _bfull(0)), dtype=jnp.int32)
        hasge = _bfull(nge) > 0
        outs = []
        for rres, vals, oc in ((tx, xv, ocx), (ty, yv, ocy), (tz, zv, ocz),
                               (tc, None, occ)):
          if vals is None:
            tot16 = cnt.astype(f32) + jnp.where(firstg, oc, zero_f)
            sfx = (_bfull(15) - lge).astype(f32)
          else:
            v16 = vals[pl.ds(i * _c(16), L)]
            s16 = plsc.cumsum(v16)
            sxv[pl.ds(8, L)] = s16
            sstart = plsc.load_gather(sxv, [8 + lane - cnt])
            tot16 = s16 - sstart + jnp.where(firstg, oc, zero_f)
            s15 = _bfull(jnp.sum(v16), f32)
            sfx = s15 - plsc.load_gather(sxv, [8 + lge])
          rres[pl.ds(i * _c(16), L)] = tot16
          outs.append(jnp.where(hasge, sfx, oc + sfx))
        ocx, ocy, ocz, occ = outs
        pos16 = i * _c(16) + lane
        ixv[pl.ds(i * _c(16), L)] = jnp.where(ge, gid16, _bfull(n) + pos16)
        anyend = anyend | jnp.where(hasge, _bfull(1), _bfull(0))
        gcur = gcur + _bfull(jnp.sum(nbi, dtype=jnp.int32))
        opengid = gcur - 1
        return (gcur, opengid, ocx, ocy, ocz, occ, anyend)

      (gcur, opengid, ocx, ocy, ocz, occ, anyend) = (
          _loop(wpt // 16, step, carry0))
      cps = [pltpu.async_copy(tx, sx_o.at[ixv], sem),
             pltpu.async_copy(ty, sy_o.at[ixv], sem),
             pltpu.async_copy(tz, sz_o.at[ixv], sem),
             pltpu.async_copy(tc, sc_o.at[ixv], sem)]
      for cp in cps:
        cp.wait()
      lastk = _bcast(kpad, L + wpt - 1)
      plsc.store_scatter(kpad, [_bfull(15)], lastk, mask=lane == 0)
      return (gcur, opengid, ocx, ocy, ocz, occ, anyend)

    (gcur, opengid, ocx, ocy, ocz, occ, anyend) = _loop(
        nwin, wstep, (gcur, opengid, ocx, ocy, ocz, occ, anyend))

    metarow = jnp.where(lane == 0, ocx,
               jnp.where(lane == 1, ocy,
                jnp.where(lane == 2, ocz,
                 jnp.where(lane == 3, occ,
                  jnp.where(lane == 4, anyend.astype(f32), zero_f)))))
    metav[...] = metarow
    pltpu.sync_copy(metav, meta.at[pl.ds(wid * _c(L), L)])

  sde = jax.ShapeDtypeStruct((nsum,), f32)
  return pl.kernel(
      body,
      out_type=(sde, sde, sde, sde,
                jax.ShapeDtypeStruct((NW * L,), f32)),
      mesh=_mesh(),
      compiler_params=pltpu.CompilerParams(needs_layout_passes=False),
      scratch_types=[
          pltpu.VMEM((wpt + 2 * L,), jnp.int32),
          pltpu.VMEM((wpt,), f32),
          pltpu.VMEM((wpt,), f32),
          pltpu.VMEM((wpt,), f32),
          pltpu.VMEM((wpt,), f32),
          pltpu.VMEM((wpt,), f32),
          pltpu.VMEM((wpt,), f32),
          pltpu.VMEM((wpt,), f32),
          pltpu.VMEM((wpt,), jnp.int32),
          pltpu.VMEM((NW * L,), jnp.int32),
          pltpu.VMEM((2 * L,), f32),
          pltpu.VMEM((L,), f32),
          pltpu.VMEM((L,), jnp.int32),
          pltpu.SemaphoreType.DMA,
      ],
  )


def _k9_means(n, nsum):
  c = n // NW
  wrow = 4096
  nwin = c // wrow
  f32 = jnp.float32

  def body(sx, sy, sz, sc, bc, meta, out, xwv, ywv, zwv, cwv, outv, bcv,
           tmv, cgv, cvv):
    wid = _wid()
    lane = _lane()
    pltpu.sync_copy(bc, bcv)
    pltpu.sync_copy(meta, tmv)
    zero_f = jnp.zeros((L,), f32)
    ng = _bfull(0)
    for w in range(NW):
      ng = ng + _bcast(bcv, L * w)
    carry = zero_f
    gb = _bfull(0)
    for w in range(NW):
      m16 = tmv[pl.ds(L * w, L)]
      trail = jnp.where(lane < 4, m16, zero_f)
      he = plsc.load_gather(tmv, [_bfull(L * w + 4)]) > 0.5
      tg = gb - 1
      cgv[pl.ds(L * w, L)] = jnp.where(he & (tg >= 0), tg, _bfull(nsum - L))
      cvv[pl.ds(L * w, L)] = jnp.where(he, carry, zero_f)
      carry = jnp.where(he, trail, carry + trail)
      gb = gb + _bcast(bcv, L * w)

    lane0 = lane == 0

    def wstep(win, _):
      rowbase = wid * _c(c) + win * _c(wrow)
      pltpu.sync_copy(sx.at[pl.ds(rowbase, wrow)], xwv)
      pltpu.sync_copy(sy.at[pl.ds(rowbase, wrow)], ywv)
      pltpu.sync_copy(sz.at[pl.ds(rowbase, wrow)], zwv)
      pltpu.sync_copy(sc.at[pl.ds(rowbase, wrow)], cwv)
      rb = _bfull(rowbase)
      for w in range(NW):
        tg = _bcast(cgv, L * w)
        rel = tg - rb
        inwin = (rel >= 0) & (rel < wrow) & lane0
        relc = jnp.maximum(rel, 0)
        plsc.addupdate_scatter(xwv, [relc], _bcast(cvv, L * w), mask=inwin)
        plsc.addupdate_scatter(ywv, [relc], _bcast(cvv, L * w + 1),
                               mask=inwin)
        plsc.addupdate_scatter(zwv, [relc], _bcast(cvv, L * w + 2),
                               mask=inwin)
        plsc.addupdate_scatter(cwv, [relc], _bcast(cvv, L * w + 3),
                               mask=inwin)

      def ostep(i, _):
        for j in range(3):
          a = 16 * j + lane
          relrow = _bfull(i * _c(16)) + a // 3
          comp = a % 3
          gx = plsc.load_gather(xwv, [relrow])
          gy = plsc.load_gather(ywv, [relrow])
          gz = plsc.load_gather(zwv, [relrow])
          val = jnp.where(comp == 0, gx, jnp.where(comp == 1, gy, gz))
          cntv = plsc.load_gather(cwv, [relrow])
          valid = (rb + relrow) < ng
          outv[pl.ds(i * _c(48) + _c(16 * j), L)] = jnp.where(
              valid, val / cntv, zero_f)
        return 0

      _loop(wrow // 16, ostep, 0)
      pltpu.sync_copy(outv, out.at[pl.ds(_c(3) * rowbase, 3 * wrow)])
      return 0

    _loop(nwin, wstep, 0)

  return pl.kernel(
      body,
      out_type=jax.ShapeDtypeStruct((3 * n,), f32),
      mesh=_mesh(),
      compiler_params=pltpu.CompilerParams(needs_layout_passes=False),
      scratch_types=[
          pltpu.VMEM((wrow,), f32),
          pltpu.VMEM((wrow,), f32),
          pltpu.VMEM((wrow,), f32),
          pltpu.VMEM((wrow,), f32),
          pltpu.VMEM((3 * wrow,), f32),
          pltpu.VMEM((NW * L,), jnp.int32),
          pltpu.VMEM((NW * L,), f32),
          pltpu.VMEM((NW * L,), jnp.int32),
          pltpu.VMEM((NW * L,), f32),
      ],
  )


def kernel(points, leaf_size):
  n = points.shape[0]
  pts_flat = points.reshape(-1)
  leaf16 = jnp.concatenate(
      [leaf_size.astype(jnp.float32),
       jnp.ones((13,), jnp.float32)])

  mm = _k1_minmax(n)(pts_flat, leaf16)
  keys, hist0 = _k2_keys_hist(n)(pts_flat, leaf16, mm)
  k1s, x1, y1, z1 = _k_permute(n, SHIFTS[0], True)(keys, hist0, pts_flat)
  h1 = _k_hist(n, SHIFTS[1])(k1s)
  k2s, x2, y2, z2 = _k_permute(n, SHIFTS[1], False)(k1s, h1, x1, y1, z1)
  h2 = _k_hist(n, SHIFTS[2])(k2s)
  k3s, x3, y3, z3 = _k_permute(n, SHIFTS[2], False)(k2s, h2, x2, y2, z2)
  bc = _k8a_bounds(n)(k3s)
  s_x, s_y, s_z, s_c, meta = _k8b_segsum(n)(k3s, x3, y3, z3, bc)
  nsum = s_x.shape[0]
  outflat = _k9_means(n, nsum)(s_x, s_y, s_z, s_c, bc, meta)
  ngroups = jnp.sum(bc.reshape(NW, L)[:, 0])
  out = outflat.reshape(n, 3)
  mask = jnp.arange(n) < ngroups
  return (out, mask)
